# Initial kernel scaffold; baseline (speedup 1.0000x reference)
import jax
import jax.numpy as jnp
from jax.experimental import pallas as pl


def kernel(x, weight, decay, commitment_cost):
    raise NotImplementedError("write your pallas kernel here")


# speedup vs baseline: 2.0055x; 2.0055x over previous
import jax
import jax.numpy as jnp
from jax.experimental import pallas as pl

_K = 1024


def _vq_kernel(xt_ref, w_ref, c_ref, q_ref, idx_ref, loss_ref):
    xp = xt_ref[0]
    w = w_ref[...]
    s = jax.lax.dot_general(
        xp, w, (((1,), (1,)), ((), ())),
        preferred_element_type=jnp.float32)
    a = jnp.sum(xp * xp, axis=1, keepdims=True)
    d = (a - 2.0 * s) + c_ref[...]
    m = jnp.min(d, axis=1, keepdims=True)
    cols = jax.lax.broadcasted_iota(jnp.int32, d.shape, 1)
    idxc = jnp.min(jnp.where(d == m, cols, _K), axis=1, keepdims=True)
    idx_ref[0] = idxc
    oh = (cols == idxc).astype(jnp.float32)
    q_ref[0] = jnp.dot(oh, w, preferred_element_type=jnp.float32)

    @pl.when(pl.program_id(0) == 0)
    def _():
        loss_ref[...] = jnp.zeros_like(loss_ref)

    loss_ref[...] += jnp.sum(m, keepdims=True)


def kernel(x, weight, decay, commitment_cost):
    b, c, h, w_ = x.shape
    p = h * w_
    xt = jnp.transpose(x, (0, 2, 3, 1)).reshape(b, p, c)
    cvec = jnp.sum(weight**2, axis=1).reshape(1, _K)
    q, idx, loss = pl.pallas_call(
        _vq_kernel,
        grid=(b,),
        in_specs=[
            pl.BlockSpec((1, p, c), lambda i: (i, 0, 0)),
            pl.BlockSpec((_K, c), lambda i: (0, 0)),
            pl.BlockSpec((1, _K), lambda i: (0, 0)),
        ],
        out_specs=[
            pl.BlockSpec((1, p, c), lambda i: (i, 0, 0)),
            pl.BlockSpec((1, p, 1), lambda i: (i, 0, 0)),
            pl.BlockSpec((1, 1), lambda i: (0, 0)),
        ],
        out_shape=[
            jax.ShapeDtypeStruct((b, p, c), jnp.float32),
            jax.ShapeDtypeStruct((b, p, 1), jnp.int32),
            jax.ShapeDtypeStruct((1, 1), jnp.float32),
        ],
    )(xt, weight, cvec)
    quantized = jnp.transpose(q.reshape(b, h, w_, c), (0, 3, 1, 2))
    embed_idx = idx.reshape(b, h, w_)
    latent_loss = commitment_cost * (loss[0, 0] / x.size)
    return (quantized, latent_loss, embed_idx)
